# baseline (speedup 1.0000x reference)
import jax
import jax.numpy as jnp
from jax import lax
from jax.experimental import pallas as pl
from jax.experimental.pallas import tpu as pltpu
from jax.experimental.pallas import tpu_sc as plsc

N = 10000
NP = 10240
E = 320000
EL = 16384
D_IN = 128
D_H = 128
D_OUT = 64

NC = 2
NS = 16
NW = NC * NS
RP = NP // NS

EP = 327680
EPT = EP // NW
CH = 128
NCH = EPT // CH

PPT = EL // NW
PCH = 128
PNCH = PPT // PCH

_mesh = plsc.VectorSubcoreMesh(
    core_axis_name="c", subcore_axis_name="s", num_cores=NC, num_subcores=NS
)


def _wid():
    return lax.axis_index("s") * NC + lax.axis_index("c")


def _rbase():
    return pl.multiple_of(lax.axis_index("s") * RP, 8)


def _deg_body(dst_hbm, zeros_hbm, ones_hbm, deg_out, idx_v, ones_v, deg_sp, sem):
    c = lax.axis_index("c")
    rb = _rbase()
    w = _wid()
    pltpu.sync_copy(zeros_hbm.at[pl.ds(rb, RP)], deg_sp.at[pl.ds(rb, RP)])
    pltpu.sync_copy(ones_hbm, ones_v)
    pltpu.sync_copy(dst_hbm.at[w], idx_v)
    plsc.subcore_barrier()

    def body(j, carry):
        pltpu.sync_copy(ones_v, deg_sp.at[idx_v.at[j]], add=True)
        return carry

    lax.fori_loop(0, NCH, body, 0)
    plsc.subcore_barrier()
    pltpu.sync_copy(deg_sp.at[pl.ds(rb, RP)], deg_out.at[c, pl.ds(rb, RP)])


_deg_kernel = pl.kernel(
    _deg_body,
    out_type=jax.ShapeDtypeStruct((NC, NP, D_H), jnp.float32),
    mesh=_mesh,
    scratch_types=[
        pltpu.VMEM((NCH, CH), jnp.int32),
        pltpu.VMEM((CH, D_H), jnp.float32),
        pltpu.VMEM_SHARED((NP, D_H), jnp.float32),
        pltpu.SemaphoreType.DMA,
    ],
)


def _agg_body(hs_hbm, src_hbm, dst_hbm, zeros_hbm, acc_out,
              sidx_v, didx_v, rows0, acc_sp, gsem0):
    c = lax.axis_index("c")
    rb = _rbase()
    w = _wid()

    @pl.when(c == 0)
    def _():
        pltpu.sync_copy(hs_hbm.at[pl.ds(rb, RP)], acc_sp.at[pl.ds(rb, RP)])

    @pl.when(c != 0)
    def _():
        pltpu.sync_copy(zeros_hbm.at[pl.ds(rb, RP)], acc_sp.at[pl.ds(rb, RP)])

    pltpu.sync_copy(src_hbm.at[w], sidx_v)
    pltpu.sync_copy(dst_hbm.at[w], didx_v)
    plsc.subcore_barrier()

    def body(j, carry):
        pltpu.async_copy(hs_hbm.at[sidx_v.at[j]], rows0, gsem0).wait()
        pltpu.sync_copy(rows0, acc_sp.at[didx_v.at[j]], add=True)
        return carry

    lax.fori_loop(0, NCH, body, 0)
    plsc.subcore_barrier()
    pltpu.sync_copy(acc_sp.at[pl.ds(rb, RP)], acc_out.at[c, pl.ds(rb, RP)])


_agg_kernel = pl.kernel(
    _agg_body,
    out_type=jax.ShapeDtypeStruct((NC, NP, D_H), jnp.float32),
    mesh=_mesh,
    scratch_types=[
        pltpu.VMEM((NCH, CH), jnp.int32),
        pltpu.VMEM((NCH, CH), jnp.int32),
        pltpu.VMEM((CH, D_H), jnp.float32),
        pltpu.VMEM_SHARED((NP, D_H), jnp.float32),
        pltpu.SemaphoreType.DMA,
    ],
)


def _pairs_body(z_hbm, sidx_hbm, didx_hbm, srows_out, drows_out,
                sidx_v, didx_v, sbuf, dbuf, sem0, sem1):
    w = _wid()
    pltpu.sync_copy(sidx_hbm.at[w], sidx_v)
    pltpu.sync_copy(didx_hbm.at[w], didx_v)

    def body(j, carry):
        ob = pl.multiple_of(w * PPT + j * PCH, 8)
        gs = pltpu.async_copy(z_hbm.at[sidx_v.at[j]], sbuf, sem0)
        gd = pltpu.async_copy(z_hbm.at[didx_v.at[j]], dbuf, sem1)
        gs.wait()
        pltpu.sync_copy(sbuf, srows_out.at[pl.ds(ob, PCH)])
        gd.wait()
        pltpu.sync_copy(dbuf, drows_out.at[pl.ds(ob, PCH)])
        return carry

    lax.fori_loop(0, PNCH, body, 0)


_pairs_kernel = pl.kernel(
    _pairs_body,
    out_type=[
        jax.ShapeDtypeStruct((EL, D_H), jnp.float32),
        jax.ShapeDtypeStruct((EL, D_H), jnp.float32),
    ],
    mesh=_mesh,
    scratch_types=[
        pltpu.VMEM((PNCH, PCH), jnp.int32),
        pltpu.VMEM((PNCH, PCH), jnp.int32),
        pltpu.VMEM((PCH, D_H), jnp.float32),
        pltpu.VMEM((PCH, D_H), jnp.float32),
        pltpu.SemaphoreType.DMA,
        pltpu.SemaphoreType.DMA,
    ],
)


_BR = 1024


DEGS = 8


def _dinv_block(degp):
    deg = degp[0, :, 0:1] + degp[1, :, 0:1] + 1.0
    return lax.rsqrt(deg)


def _tc_mm_body(x_ref, w_ref, out_ref):
    out_ref[...] = jnp.dot(x_ref[...], w_ref[...],
                           preferred_element_type=jnp.float32)


def _tc_scale_body(h_ref, degp_ref, out_ref):
    out_ref[...] = h_ref[...] * _dinv_block(degp_ref[...])


def _tc_mid_body(acc_ref, degp_ref, b1_ref, w2_ref, out_ref):
    dinv = _dinv_block(degp_ref[...])
    z1 = jnp.maximum((acc_ref[0] + acc_ref[1]) * dinv + b1_ref[...], 0.0)
    out_ref[...] = jnp.dot(z1, w2_ref[...], preferred_element_type=jnp.float32) * dinv


def _tc_z2_body(acc_ref, degp_ref, b2_ref, out_ref):
    dinv = _dinv_block(degp_ref[...])
    out_ref[...] = (acc_ref[0] + acc_ref[1]) * dinv + b2_ref[...]


def _tc_dot_body(s_ref, d_ref, out_ref):
    out_ref[...] = jnp.sum(s_ref[...] * d_ref[...], axis=1, keepdims=True)


def _degp_spec():
    return pl.BlockSpec((NC, _BR, DEGS), lambda i: (0, i, 0))


def _row_spec(d):
    return pl.BlockSpec((_BR, d), lambda i: (i, 0))


def _acc_spec(d):
    return pl.BlockSpec((NC, _BR, d), lambda i: (0, i, 0))


def _full_spec(shape):
    return pl.BlockSpec(shape, lambda i: tuple(0 for _ in shape))


_tc_mm = pl.pallas_call(
    _tc_mm_body,
    grid=(NP // _BR,),
    in_specs=[_row_spec(D_IN), _full_spec((D_IN, D_H))],
    out_specs=_row_spec(D_H),
    out_shape=jax.ShapeDtypeStruct((NP, D_H), jnp.float32),
)

_tc_scale = pl.pallas_call(
    _tc_scale_body,
    grid=(NP // _BR,),
    in_specs=[_row_spec(D_H), _degp_spec()],
    out_specs=_row_spec(D_H),
    out_shape=jax.ShapeDtypeStruct((NP, D_H), jnp.float32),
)

_tc_mid = pl.pallas_call(
    _tc_mid_body,
    grid=(NP // _BR,),
    in_specs=[_acc_spec(D_H), _degp_spec(), _full_spec((1, D_H)),
              _full_spec((D_H, D_H))],
    out_specs=_row_spec(D_H),
    out_shape=jax.ShapeDtypeStruct((NP, D_H), jnp.float32),
)

_tc_z2 = pl.pallas_call(
    _tc_z2_body,
    grid=(NP // _BR,),
    in_specs=[_acc_spec(D_H), _degp_spec(), _full_spec((1, D_H))],
    out_specs=_row_spec(D_H),
    out_shape=jax.ShapeDtypeStruct((NP, D_H), jnp.float32),
)

_DBR = 2048

_tc_dot = pl.pallas_call(
    _tc_dot_body,
    grid=(EL // _DBR,),
    in_specs=[pl.BlockSpec((_DBR, D_H), lambda i: (i, 0)),
              pl.BlockSpec((_DBR, D_H), lambda i: (i, 0))],
    out_specs=pl.BlockSpec((_DBR, 1), lambda i: (i, 0)),
    out_shape=jax.ShapeDtypeStruct((EL, 1), jnp.float32),
)


@jax.jit
def kernel(x, edge_index, edge_label_index, W1, b1, W2, b2):
    epad = jnp.full((2, EP - E), N, jnp.int32)
    ei_p = jnp.concatenate([edge_index, epad], axis=1)
    src_r = ei_p[0].reshape(NW, NCH, CH)
    dst_r = ei_p[1].reshape(NW, NCH, CH)
    els_r = edge_label_index[0].reshape(NW, PNCH, PCH)
    eld_r = edge_label_index[1].reshape(NW, PNCH, PCH)

    xp = jnp.concatenate([x, jnp.zeros((NP - N, D_IN), x.dtype)], axis=0)
    w2p = jnp.concatenate(
        [W2, jnp.zeros((D_H, D_H - D_OUT), W2.dtype)], axis=1)
    b2p = jnp.concatenate([b2, jnp.zeros((D_H - D_OUT,), b2.dtype)])

    zeros_h = jnp.zeros((NP, D_H), jnp.float32)
    ones_ch = jnp.ones((CH, D_H), jnp.float32)

    h1 = _tc_mm(xp, W1)
    degp = _deg_kernel(dst_r, zeros_h, ones_ch)
    degs = degp[:, :, :DEGS]

    hs1 = _tc_scale(h1, degs)
    acc1 = _agg_kernel(hs1, src_r, dst_r, zeros_h)
    hs2 = _tc_mid(acc1, degs, b1.reshape(1, D_H), w2p)
    acc2 = _agg_kernel(hs2, src_r, dst_r, zeros_h)
    z2 = _tc_z2(acc2, degs, b2p.reshape(1, D_H))

    srows, drows = _pairs_kernel(z2, els_r, eld_r)
    return _tc_dot(srows, drows).reshape(-1)


# speedup vs baseline: 2.0200x; 2.0200x over previous
import jax
import jax.numpy as jnp
from jax import lax
from jax.experimental import pallas as pl
from jax.experimental.pallas import tpu as pltpu
from jax.experimental.pallas import tpu_sc as plsc

N = 10000
NP = 10240
E = 320000
EL = 16384
D_IN = 128
D_H = 128
D_OUT = 64

NC = 2
NS = 16
NW = NC * NS
RP = NP // NS

EP = 327680
EPT = EP // NW
CH = 128
NCH = EPT // CH

PPT = EL // NW
PCH = 128
PNCH = PPT // PCH

_mesh = plsc.VectorSubcoreMesh(
    core_axis_name="c", subcore_axis_name="s", num_cores=NC, num_subcores=NS
)


def _wid():
    return lax.axis_index("s") * NC + lax.axis_index("c")


def _rbase():
    return pl.multiple_of(lax.axis_index("s") * RP, 8)


def _deg_body(dst_hbm, zeros_hbm, ones_hbm, deg_out, idx_v, ones_v, deg_sp, sem):
    c = lax.axis_index("c")
    rb = _rbase()
    w = _wid()
    pltpu.sync_copy(zeros_hbm.at[pl.ds(rb, RP)], deg_sp.at[pl.ds(rb, RP)])
    pltpu.sync_copy(ones_hbm, ones_v)
    pltpu.sync_copy(dst_hbm.at[w], idx_v)
    plsc.subcore_barrier()

    def body(j, carry):
        pltpu.sync_copy(ones_v, deg_sp.at[idx_v.at[j]], add=True)
        return carry

    lax.fori_loop(0, NCH, body, 0)
    plsc.subcore_barrier()
    pltpu.sync_copy(deg_sp.at[pl.ds(rb, RP)], deg_out.at[c, pl.ds(rb, RP)])


_deg_kernel = pl.kernel(
    _deg_body,
    out_type=jax.ShapeDtypeStruct((NC, NP, D_H), jnp.float32),
    mesh=_mesh,
    scratch_types=[
        pltpu.VMEM((NCH, CH), jnp.int32),
        pltpu.VMEM((CH, D_H), jnp.float32),
        pltpu.VMEM_SHARED((NP, D_H), jnp.float32),
        pltpu.SemaphoreType.DMA,
    ],
)


def _agg_body(hs_hbm, src_hbm, dst_hbm, zeros_hbm, acc_out,
              sidx_v, didx_v, rows0, acc_sp, gsem0):
    c = lax.axis_index("c")
    rb = _rbase()
    w = _wid()

    @pl.when(c == 0)
    def _():
        pltpu.sync_copy(hs_hbm.at[pl.ds(rb, RP)], acc_sp.at[pl.ds(rb, RP)])

    @pl.when(c != 0)
    def _():
        pltpu.sync_copy(zeros_hbm.at[pl.ds(rb, RP)], acc_sp.at[pl.ds(rb, RP)])

    pltpu.sync_copy(src_hbm.at[w], sidx_v)
    pltpu.sync_copy(dst_hbm.at[w], didx_v)
    plsc.subcore_barrier()

    def body(j, carry):
        pltpu.async_copy(hs_hbm.at[sidx_v.at[j]], rows0, gsem0).wait()
        pltpu.sync_copy(rows0, acc_sp.at[didx_v.at[j]], add=True)
        return carry

    lax.fori_loop(0, NCH, body, 0)
    plsc.subcore_barrier()
    pltpu.sync_copy(acc_sp.at[pl.ds(rb, RP)], acc_out.at[c, pl.ds(rb, RP)])


_agg_kernel = pl.kernel(
    _agg_body,
    out_type=jax.ShapeDtypeStruct((NC, NP, D_H), jnp.float32),
    mesh=_mesh,
    scratch_types=[
        pltpu.VMEM((NCH, CH), jnp.int32),
        pltpu.VMEM((NCH, CH), jnp.int32),
        pltpu.VMEM((CH, D_H), jnp.float32),
        pltpu.VMEM_SHARED((NP, D_H), jnp.float32),
        pltpu.SemaphoreType.DMA,
    ],
)


def _pairs_body(z_hbm, sidx_hbm, didx_hbm, srows_out, drows_out,
                sidx_v, didx_v, sbuf, dbuf, sem0, sem1):
    w = _wid()
    pltpu.sync_copy(sidx_hbm.at[w], sidx_v)
    pltpu.sync_copy(didx_hbm.at[w], didx_v)

    def body(j, carry):
        ob = pl.multiple_of(w * PPT + j * PCH, 8)
        gs = pltpu.async_copy(z_hbm.at[sidx_v.at[j]], sbuf, sem0)
        gd = pltpu.async_copy(z_hbm.at[didx_v.at[j]], dbuf, sem1)
        gs.wait()
        pltpu.sync_copy(sbuf, srows_out.at[pl.ds(ob, PCH)])
        gd.wait()
        pltpu.sync_copy(dbuf, drows_out.at[pl.ds(ob, PCH)])
        return carry

    lax.fori_loop(0, PNCH, body, 0)


_pairs_kernel = pl.kernel(
    _pairs_body,
    out_type=[
        jax.ShapeDtypeStruct((EL, D_H), jnp.float32),
        jax.ShapeDtypeStruct((EL, D_H), jnp.float32),
    ],
    mesh=_mesh,
    scratch_types=[
        pltpu.VMEM((PNCH, PCH), jnp.int32),
        pltpu.VMEM((PNCH, PCH), jnp.int32),
        pltpu.VMEM((PCH, D_H), jnp.float32),
        pltpu.VMEM((PCH, D_H), jnp.float32),
        pltpu.SemaphoreType.DMA,
        pltpu.SemaphoreType.DMA,
    ],
)


_BR = 1024


DEGS = 8


def _dinv_block(degp):
    deg = degp[0, :, 0:1] + degp[1, :, 0:1] + 1.0
    return lax.rsqrt(deg)


def _tc_mm_body(x_ref, w_ref, out_ref):
    out_ref[...] = jnp.dot(x_ref[...], w_ref[...],
                           preferred_element_type=jnp.float32)


def _tc_scale_body(h_ref, degp_ref, out_ref):
    out_ref[...] = h_ref[...] * _dinv_block(degp_ref[...])


def _tc_mid_body(acc_ref, degp_ref, b1_ref, w2_ref, out_ref):
    dinv = _dinv_block(degp_ref[...])
    z1 = jnp.maximum((acc_ref[0] + acc_ref[1]) * dinv + b1_ref[...], 0.0)
    out_ref[...] = jnp.dot(z1, w2_ref[...], preferred_element_type=jnp.float32) * dinv


def _tc_z2_body(acc_ref, degp_ref, b2_ref, out_ref):
    dinv = _dinv_block(degp_ref[...])
    out_ref[...] = (acc_ref[0] + acc_ref[1]) * dinv + b2_ref[...]


def _tc_dot_body(s_ref, d_ref, out_ref):
    out_ref[...] = jnp.sum(s_ref[...] * d_ref[...], axis=1, keepdims=True)


def _degp_spec():
    return pl.BlockSpec((NC, _BR, DEGS), lambda i: (0, i, 0))


def _row_spec(d):
    return pl.BlockSpec((_BR, d), lambda i: (i, 0))


def _acc_spec(d):
    return pl.BlockSpec((NC, _BR, d), lambda i: (0, i, 0))


def _full_spec(shape):
    return pl.BlockSpec(shape, lambda i: tuple(0 for _ in shape))


_tc_mm = pl.pallas_call(
    _tc_mm_body,
    grid=(NP // _BR,),
    in_specs=[_row_spec(D_IN), _full_spec((D_IN, D_H))],
    out_specs=_row_spec(D_H),
    out_shape=jax.ShapeDtypeStruct((NP, D_H), jnp.float32),
)

_tc_scale = pl.pallas_call(
    _tc_scale_body,
    grid=(NP // _BR,),
    in_specs=[_row_spec(D_H), _degp_spec()],
    out_specs=_row_spec(D_H),
    out_shape=jax.ShapeDtypeStruct((NP, D_H), jnp.float32),
)

_tc_mid = pl.pallas_call(
    _tc_mid_body,
    grid=(NP // _BR,),
    in_specs=[_acc_spec(D_H), _degp_spec(), _full_spec((1, D_H)),
              _full_spec((D_H, D_H))],
    out_specs=_row_spec(D_H),
    out_shape=jax.ShapeDtypeStruct((NP, D_H), jnp.float32),
)

_tc_z2 = pl.pallas_call(
    _tc_z2_body,
    grid=(NP // _BR,),
    in_specs=[_acc_spec(D_H), _degp_spec(), _full_spec((1, D_H))],
    out_specs=_row_spec(D_H),
    out_shape=jax.ShapeDtypeStruct((NP, D_H), jnp.float32),
)

_DBR = 2048

_tc_dot = pl.pallas_call(
    _tc_dot_body,
    grid=(EL // _DBR,),
    in_specs=[pl.BlockSpec((_DBR, D_H), lambda i: (i, 0)),
              pl.BlockSpec((_DBR, D_H), lambda i: (i, 0))],
    out_specs=pl.BlockSpec((_DBR, 1), lambda i: (i, 0)),
    out_shape=jax.ShapeDtypeStruct((EL, 1), jnp.float32),
)


@jax.jit
def kernel(x, edge_index, edge_label_index, W1, b1, W2, b2):
    pad_idx = N + jnp.arange(EP - E, dtype=jnp.int32) % (NP - N)
    epad = jnp.stack([pad_idx, pad_idx])
    ei_p = jnp.concatenate([edge_index, epad], axis=1)
    src_r = ei_p[0].reshape(NW, NCH, CH)
    dst_r = ei_p[1].reshape(NW, NCH, CH)
    els_r = edge_label_index[0].reshape(NW, PNCH, PCH)
    eld_r = edge_label_index[1].reshape(NW, PNCH, PCH)

    xp = jnp.concatenate([x, jnp.zeros((NP - N, D_IN), x.dtype)], axis=0)
    w2p = jnp.concatenate(
        [W2, jnp.zeros((D_H, D_H - D_OUT), W2.dtype)], axis=1)
    b2p = jnp.concatenate([b2, jnp.zeros((D_H - D_OUT,), b2.dtype)])

    zeros_h = jnp.zeros((NP, D_H), jnp.float32)
    ones_ch = jnp.ones((CH, D_H), jnp.float32)

    h1 = _tc_mm(xp, W1)
    degp = _deg_kernel(dst_r, zeros_h, ones_ch)
    degs = degp[:, :, :DEGS]

    hs1 = _tc_scale(h1, degs)
    acc1 = _agg_kernel(hs1, src_r, dst_r, zeros_h)
    hs2 = _tc_mid(acc1, degs, b1.reshape(1, D_H), w2p)
    acc2 = _agg_kernel(hs2, src_r, dst_r, zeros_h)
    z2 = _tc_z2(acc2, degs, b2p.reshape(1, D_H))

    srows, drows = _pairs_kernel(z2, els_r, eld_r)
    return _tc_dot(srows, drows).reshape(-1)


# speedup vs baseline: 2.0861x; 1.0327x over previous
import jax
import jax.numpy as jnp
from jax import lax
from jax.experimental import pallas as pl
from jax.experimental.pallas import tpu as pltpu
from jax.experimental.pallas import tpu_sc as plsc

N = 10000
NP = 10240
E = 320000
EL = 16384
D_IN = 128
D_H = 128
D_OUT = 64

NC = 2
NS = 16
NW = NC * NS
RP = NP // NS

EP = 327680
EPT = EP // NW
CH = 128
NCH = EPT // CH

NH = NP // 2

PPT = EL // NW
PCH = 128
PNCH = PPT // PCH

_mesh = plsc.VectorSubcoreMesh(
    core_axis_name="c", subcore_axis_name="s", num_cores=NC, num_subcores=NS
)


def _wid():
    return lax.axis_index("s") * NC + lax.axis_index("c")


def _rbase():
    return pl.multiple_of(lax.axis_index("s") * RP, 8)


def _deg_body(dst_hbm, zeros_hbm, ones_hbm, deg_out, idx_v, ones_v, deg_sp, sem):
    c = lax.axis_index("c")
    rb = _rbase()
    w = _wid()
    pltpu.sync_copy(zeros_hbm.at[pl.ds(rb, RP)], deg_sp.at[pl.ds(rb, RP)])
    pltpu.sync_copy(ones_hbm, ones_v)
    pltpu.sync_copy(dst_hbm.at[w], idx_v)
    plsc.subcore_barrier()

    def body(j, carry):
        pltpu.sync_copy(ones_v, deg_sp.at[idx_v.at[j]], add=True)
        return carry

    lax.fori_loop(0, NCH, body, 0)
    plsc.subcore_barrier()
    pltpu.sync_copy(deg_sp.at[pl.ds(rb, RP)], deg_out.at[c, pl.ds(rb, RP)])


_deg_kernel = pl.kernel(
    _deg_body,
    out_type=jax.ShapeDtypeStruct((NC, NP, D_H), jnp.float32),
    mesh=_mesh,
    scratch_types=[
        pltpu.VMEM((NCH, CH), jnp.int32),
        pltpu.VMEM((CH, D_H), jnp.float32),
        pltpu.VMEM_SHARED((NP, D_H), jnp.float32),
        pltpu.SemaphoreType.DMA,
    ],
)


def _agg_body(hs_hbm, src_hbm, dst_hbm, zeros_hbm, acc_out,
              sidx_v, didx_v, rows0, acc_sp, gsem0):
    c = lax.axis_index("c")
    rb = _rbase()
    w = _wid()

    @pl.when(c == 0)
    def _():
        pltpu.sync_copy(hs_hbm.at[pl.ds(rb, RP)], acc_sp.at[pl.ds(rb, RP)])

    @pl.when(c != 0)
    def _():
        pltpu.sync_copy(zeros_hbm.at[pl.ds(rb, RP)], acc_sp.at[pl.ds(rb, RP)])

    pltpu.sync_copy(src_hbm.at[w], sidx_v)
    pltpu.sync_copy(dst_hbm.at[w], didx_v)
    plsc.subcore_barrier()

    def body(j, carry):
        pltpu.async_copy(hs_hbm.at[sidx_v.at[j]], rows0, gsem0).wait()
        pltpu.sync_copy(rows0, acc_sp.at[didx_v.at[j]], add=True)
        return carry

    lax.fori_loop(0, NCH, body, 0)
    plsc.subcore_barrier()
    pltpu.sync_copy(acc_sp.at[pl.ds(rb, RP)], acc_out.at[c, pl.ds(rb, RP)])


_agg_kernel = pl.kernel(
    _agg_body,
    out_type=jax.ShapeDtypeStruct((NC, NP, D_H), jnp.float32),
    mesh=_mesh,
    scratch_types=[
        pltpu.VMEM((NCH, CH), jnp.int32),
        pltpu.VMEM((NCH, CH), jnp.int32),
        pltpu.VMEM((CH, D_H), jnp.float32),
        pltpu.VMEM_SHARED((NP, D_H), jnp.float32),
        pltpu.SemaphoreType.DMA,
    ],
)


def _agg2_body(hs_hbm, src_hbm, dst_hbm, zeros_hbm, acc_out,
               sidx_v, didx_v, rows0, rows1, acc_sp, gsem0, gsem1):
    c = lax.axis_index("c")
    s = lax.axis_index("s")
    rb = pl.multiple_of(s * (NH // NS), 8)
    w = _wid()
    hp = NH // NS

    pltpu.sync_copy(zeros_hbm.at[pl.ds(rb, hp)], acc_sp.at[pl.ds(rb, hp)])
    pltpu.sync_copy(src_hbm.at[w], sidx_v)
    pltpu.sync_copy(dst_hbm.at[w], didx_v)
    plsc.subcore_barrier()

    def body(jj, carry):
        t = jj * 2
        g0 = pltpu.async_copy(hs_hbm.at[sidx_v.at[t]], rows0, gsem0)
        g1 = pltpu.async_copy(hs_hbm.at[sidx_v.at[t + 1]], rows1, gsem1)
        g0.wait()
        pltpu.sync_copy(rows0, acc_sp.at[didx_v.at[t]], add=True)
        g1.wait()
        pltpu.sync_copy(rows1, acc_sp.at[didx_v.at[t + 1]], add=True)
        return carry

    lax.fori_loop(0, NCH // 2, body, 0)
    plsc.subcore_barrier()
    pltpu.sync_copy(acc_sp.at[pl.ds(rb, hp)], acc_out.at[c, pl.ds(rb, hp)])


_agg2_kernel = pl.kernel(
    _agg2_body,
    out_type=jax.ShapeDtypeStruct((NC, NH, D_H), jnp.float32),
    mesh=_mesh,
    scratch_types=[
        pltpu.VMEM((NCH, CH), jnp.int32),
        pltpu.VMEM((NCH, CH), jnp.int32),
        pltpu.VMEM((CH, D_H), jnp.float32),
        pltpu.VMEM((CH, D_H), jnp.float32),
        pltpu.VMEM_SHARED((NH, D_H), jnp.float32),
        pltpu.SemaphoreType.DMA,
        pltpu.SemaphoreType.DMA,
    ],
)


def _pairs_body(z_hbm, sidx_hbm, didx_hbm, srows_out, drows_out,
                sidx_v, didx_v, sbuf, dbuf, sem0, sem1):
    w = _wid()
    pltpu.sync_copy(sidx_hbm.at[w], sidx_v)
    pltpu.sync_copy(didx_hbm.at[w], didx_v)

    def body(j, carry):
        ob = pl.multiple_of(w * PPT + j * PCH, 8)
        gs = pltpu.async_copy(z_hbm.at[sidx_v.at[j]], sbuf, sem0)
        gd = pltpu.async_copy(z_hbm.at[didx_v.at[j]], dbuf, sem1)
        gs.wait()
        pltpu.sync_copy(sbuf, srows_out.at[pl.ds(ob, PCH)])
        gd.wait()
        pltpu.sync_copy(dbuf, drows_out.at[pl.ds(ob, PCH)])
        return carry

    lax.fori_loop(0, PNCH, body, 0)


_pairs_kernel = pl.kernel(
    _pairs_body,
    out_type=[
        jax.ShapeDtypeStruct((EL, D_H), jnp.float32),
        jax.ShapeDtypeStruct((EL, D_H), jnp.float32),
    ],
    mesh=_mesh,
    scratch_types=[
        pltpu.VMEM((PNCH, PCH), jnp.int32),
        pltpu.VMEM((PNCH, PCH), jnp.int32),
        pltpu.VMEM((PCH, D_H), jnp.float32),
        pltpu.VMEM((PCH, D_H), jnp.float32),
        pltpu.SemaphoreType.DMA,
        pltpu.SemaphoreType.DMA,
    ],
)


_BR = 1024


DEGS = 8


def _dinv_block(degp):
    deg = degp[0, :, 0:1] + degp[1, :, 0:1] + 1.0
    return lax.rsqrt(deg)


def _tc_mm_body(x_ref, w_ref, out_ref):
    out_ref[...] = jnp.dot(x_ref[...], w_ref[...],
                           preferred_element_type=jnp.float32)


def _tc_scale_body(h_ref, degp_ref, out_ref):
    out_ref[...] = h_ref[...] * _dinv_block(degp_ref[...])


def _tc_mid_body(acc_ref, degp_ref, b1_ref, w2_ref, out_ref):
    dinv = _dinv_block(degp_ref[...])
    z1 = jnp.maximum((acc_ref[0] + acc_ref[1]) * dinv + b1_ref[...], 0.0)
    v = jnp.dot(z1, w2_ref[...], preferred_element_type=jnp.float32) * dinv
    out_ref[0] = v
    out_ref[1] = jnp.concatenate(
        [jnp.zeros((_BR, D_OUT), jnp.float32), v[:, :D_OUT]], axis=1)


def _tc_z2_body(acc_ref, degp_ref, hself_ref, b2_ref, out_ref):
    dinv = _dinv_block(degp_ref[...])
    v = (acc_ref[0] + acc_ref[1] + hself_ref[...]) * dinv + b2_ref[...]
    out_ref[...] = jnp.concatenate(
        [v, jnp.zeros((_BR, D_H - D_OUT), jnp.float32)], axis=1)


def _tc_dot_body(s_ref, d_ref, out_ref):
    out_ref[...] = jnp.sum(s_ref[...] * d_ref[...], axis=1, keepdims=True)


def _degp_spec():
    return pl.BlockSpec((NC, _BR, DEGS), lambda i: (0, i, 0))


def _row_spec(d):
    return pl.BlockSpec((_BR, d), lambda i: (i, 0))


def _acc_spec(d):
    return pl.BlockSpec((NC, _BR, d), lambda i: (0, i, 0))


def _full_spec(shape):
    return pl.BlockSpec(shape, lambda i: tuple(0 for _ in shape))


_tc_mm = pl.pallas_call(
    _tc_mm_body,
    grid=(NP // _BR,),
    in_specs=[_row_spec(D_IN), _full_spec((D_IN, D_H))],
    out_specs=_row_spec(D_H),
    out_shape=jax.ShapeDtypeStruct((NP, D_H), jnp.float32),
)

_tc_scale = pl.pallas_call(
    _tc_scale_body,
    grid=(NP // _BR,),
    in_specs=[_row_spec(D_H), _degp_spec()],
    out_specs=_row_spec(D_H),
    out_shape=jax.ShapeDtypeStruct((NP, D_H), jnp.float32),
)

_tc_mid = pl.pallas_call(
    _tc_mid_body,
    grid=(NP // _BR,),
    in_specs=[_acc_spec(D_H), _degp_spec(), _full_spec((1, D_H)),
              _full_spec((D_H, D_H))],
    out_specs=pl.BlockSpec((NC, _BR, D_H), lambda i: (0, i, 0)),
    out_shape=jax.ShapeDtypeStruct((NC, NP, D_H), jnp.float32),
)

_tc_z2 = pl.pallas_call(
    _tc_z2_body,
    grid=(NP // _BR,),
    in_specs=[_acc_spec(D_OUT), _degp_spec(), _row_spec(D_OUT),
              _full_spec((1, D_OUT))],
    out_specs=_row_spec(D_H),
    out_shape=jax.ShapeDtypeStruct((NP, D_H), jnp.float32),
)

_DBR = 2048

_tc_dot = pl.pallas_call(
    _tc_dot_body,
    grid=(EL // _DBR,),
    in_specs=[pl.BlockSpec((_DBR, D_H), lambda i: (i, 0)),
              pl.BlockSpec((_DBR, D_H), lambda i: (i, 0))],
    out_specs=pl.BlockSpec((_DBR, 1), lambda i: (i, 0)),
    out_shape=jax.ShapeDtypeStruct((EL, 1), jnp.float32),
)


@jax.jit
def kernel(x, edge_index, edge_label_index, W1, b1, W2, b2):
    pad_idx = N + jnp.arange(EP - E, dtype=jnp.int32) % (NP - N)
    epad = jnp.stack([pad_idx, pad_idx])
    ei_p = jnp.concatenate([edge_index, epad], axis=1)
    src_r = ei_p[0].reshape(NW, NCH, CH)
    dst_r = ei_p[1].reshape(NW, NCH, CH)
    els_r = edge_label_index[0].reshape(NW, PNCH, PCH)
    eld_r = edge_label_index[1].reshape(NW, PNCH, PCH)

    xp = jnp.concatenate([x, jnp.zeros((NP - N, D_IN), x.dtype)], axis=0)
    w2p = jnp.concatenate(
        [W2, jnp.zeros((D_H, D_H - D_OUT), W2.dtype)], axis=1)
    b2p = jnp.concatenate([b2, jnp.zeros((D_H - D_OUT,), b2.dtype)])

    zeros_h = jnp.zeros((NP, D_H), jnp.float32)
    ones_ch = jnp.ones((CH, D_H), jnp.float32)

    h1 = _tc_mm(xp, W1)
    degp = _deg_kernel(dst_r, zeros_h, ones_ch)
    degs = degp[:, :, :DEGS]

    hs1 = _tc_scale(h1, degs)
    acc1 = _agg_kernel(hs1, src_r, dst_r, zeros_h)
    hs2d = _tc_mid(acc1, degs, b1.reshape(1, D_H), w2p)
    g2 = (ei_p[0] + NP * (ei_p[1] & 1)).reshape(NW, NCH, CH)
    d2 = (ei_p[1] >> 1).reshape(NW, NCH, CH)
    zeros_half = jnp.zeros((NH, D_H), jnp.float32)
    acc2p = _agg2_kernel(hs2d.reshape(2 * NP, D_H), g2, d2, zeros_half)
    acc2 = acc2p.reshape(NC, NP, D_OUT)
    hs2self = hs2d[0, :, :D_OUT]
    z2 = _tc_z2(acc2, degs, hs2self, b2p[:D_OUT].reshape(1, D_OUT))

    srows, drows = _pairs_kernel(z2, els_r, eld_r)
    return _tc_dot(srows, drows).reshape(-1)


# speedup vs baseline: 2.1096x; 1.0113x over previous
import jax
import jax.numpy as jnp
from jax import lax
from jax.experimental import pallas as pl
from jax.experimental.pallas import tpu as pltpu
from jax.experimental.pallas import tpu_sc as plsc

N = 10000
NP = 10240
E = 320000
EL = 16384
D_IN = 128
D_H = 128
D_OUT = 64

NC = 2
NS = 16
NW = NC * NS
RP = NP // NS

EP = 327680
EPT = EP // NW
CH = 128
NCH = EPT // CH

NH = NP // 2

PPT = EL // NW
PCH = 128
PNCH = PPT // PCH

_mesh = plsc.VectorSubcoreMesh(
    core_axis_name="c", subcore_axis_name="s", num_cores=NC, num_subcores=NS
)


def _wid():
    return lax.axis_index("s") * NC + lax.axis_index("c")


def _rbase():
    return pl.multiple_of(lax.axis_index("s") * RP, 8)


def _deg_body(dst_hbm, zeros_hbm, ones_hbm, deg_out, idx_v, ones_v, deg_sp, sem):
    c = lax.axis_index("c")
    rb = _rbase()
    w = _wid()
    pltpu.sync_copy(zeros_hbm.at[pl.ds(rb, RP)], deg_sp.at[pl.ds(rb, RP)])
    pltpu.sync_copy(ones_hbm, ones_v)
    pltpu.sync_copy(dst_hbm.at[w], idx_v)
    plsc.subcore_barrier()

    def body(j, carry):
        pltpu.sync_copy(ones_v, deg_sp.at[idx_v.at[j]], add=True)
        return carry

    lax.fori_loop(0, NCH, body, 0)
    plsc.subcore_barrier()
    pltpu.sync_copy(deg_sp.at[pl.ds(rb, RP)], deg_out.at[c, pl.ds(rb, RP)])


_deg_kernel = pl.kernel(
    _deg_body,
    out_type=jax.ShapeDtypeStruct((NC, NP, D_H), jnp.float32),
    mesh=_mesh,
    scratch_types=[
        pltpu.VMEM((NCH, CH), jnp.int32),
        pltpu.VMEM((CH, D_H), jnp.float32),
        pltpu.VMEM_SHARED((NP, D_H), jnp.float32),
        pltpu.SemaphoreType.DMA,
    ],
)


def _agg_body(hs_hbm, src_hbm, dst_hbm, zeros_hbm, acc_out,
              sidx_v, didx_v, rows0, acc_sp, gsem0):
    c = lax.axis_index("c")
    rb = _rbase()
    w = _wid()

    @pl.when(c == 0)
    def _():
        pltpu.sync_copy(hs_hbm.at[pl.ds(rb, RP)], acc_sp.at[pl.ds(rb, RP)])

    @pl.when(c != 0)
    def _():
        pltpu.sync_copy(zeros_hbm.at[pl.ds(rb, RP)], acc_sp.at[pl.ds(rb, RP)])

    pltpu.sync_copy(src_hbm.at[w], sidx_v)
    pltpu.sync_copy(dst_hbm.at[w], didx_v)
    plsc.subcore_barrier()

    def body(j, carry):
        pltpu.async_copy(hs_hbm.at[sidx_v.at[j]], rows0, gsem0).wait()
        pltpu.sync_copy(rows0, acc_sp.at[didx_v.at[j]], add=True)
        return carry

    lax.fori_loop(0, NCH, body, 0)
    plsc.subcore_barrier()
    pltpu.sync_copy(acc_sp.at[pl.ds(rb, RP)], acc_out.at[c, pl.ds(rb, RP)])


_agg_kernel = pl.kernel(
    _agg_body,
    out_type=jax.ShapeDtypeStruct((NC, NP, D_H), jnp.float32),
    mesh=_mesh,
    scratch_types=[
        pltpu.VMEM((NCH, CH), jnp.int32),
        pltpu.VMEM((NCH, CH), jnp.int32),
        pltpu.VMEM((CH, D_H), jnp.float32),
        pltpu.VMEM_SHARED((NP, D_H), jnp.float32),
        pltpu.SemaphoreType.DMA,
    ],
)


def _agg2_body(hs_hbm, src_hbm, dst_hbm, zeros_hbm, acc_out,
               sidx_v, didx_v, rows0, rows1, rows2, rows3, acc_sp,
               gsem0, gsem1, gsem2, gsem3):
    c = lax.axis_index("c")
    s = lax.axis_index("s")
    rb = pl.multiple_of(s * (NH // NS), 8)
    w = _wid()
    hp = NH // NS

    pltpu.sync_copy(zeros_hbm.at[pl.ds(rb, hp)], acc_sp.at[pl.ds(rb, hp)])
    pltpu.sync_copy(src_hbm.at[w], sidx_v)
    pltpu.sync_copy(dst_hbm.at[w], didx_v)
    plsc.subcore_barrier()

    def body(jj, carry):
        t = jj * 4
        g0 = pltpu.async_copy(hs_hbm.at[sidx_v.at[t]], rows0, gsem0)
        g1 = pltpu.async_copy(hs_hbm.at[sidx_v.at[t + 1]], rows1, gsem1)
        g2 = pltpu.async_copy(hs_hbm.at[sidx_v.at[t + 2]], rows2, gsem2)
        g3 = pltpu.async_copy(hs_hbm.at[sidx_v.at[t + 3]], rows3, gsem3)
        g0.wait()
        pltpu.sync_copy(rows0, acc_sp.at[didx_v.at[t]], add=True)
        g1.wait()
        pltpu.sync_copy(rows1, acc_sp.at[didx_v.at[t + 1]], add=True)
        g2.wait()
        pltpu.sync_copy(rows2, acc_sp.at[didx_v.at[t + 2]], add=True)
        g3.wait()
        pltpu.sync_copy(rows3, acc_sp.at[didx_v.at[t + 3]], add=True)
        return carry

    lax.fori_loop(0, NCH // 4, body, 0)
    plsc.subcore_barrier()
    pltpu.sync_copy(acc_sp.at[pl.ds(rb, hp)], acc_out.at[c, pl.ds(rb, hp)])


_agg2_kernel = pl.kernel(
    _agg2_body,
    out_type=jax.ShapeDtypeStruct((NC, NH, D_H), jnp.float32),
    mesh=_mesh,
    scratch_types=[
        pltpu.VMEM((NCH, CH), jnp.int32),
        pltpu.VMEM((NCH, CH), jnp.int32),
        pltpu.VMEM((CH, D_H), jnp.float32),
        pltpu.VMEM((CH, D_H), jnp.float32),
        pltpu.VMEM((CH, D_H), jnp.float32),
        pltpu.VMEM((CH, D_H), jnp.float32),
        pltpu.VMEM_SHARED((NH, D_H), jnp.float32),
        pltpu.SemaphoreType.DMA,
        pltpu.SemaphoreType.DMA,
        pltpu.SemaphoreType.DMA,
        pltpu.SemaphoreType.DMA,
    ],
)


def _pairs_body(z_hbm, sidx_hbm, didx_hbm, srows_out, drows_out,
                sidx_v, didx_v, sbuf, dbuf, sem0, sem1):
    w = _wid()
    pltpu.sync_copy(sidx_hbm.at[w], sidx_v)
    pltpu.sync_copy(didx_hbm.at[w], didx_v)

    def body(j, carry):
        ob = pl.multiple_of(w * PPT + j * PCH, 8)
        gs = pltpu.async_copy(z_hbm.at[sidx_v.at[j]], sbuf, sem0)
        gd = pltpu.async_copy(z_hbm.at[didx_v.at[j]], dbuf, sem1)
        gs.wait()
        pltpu.sync_copy(sbuf, srows_out.at[pl.ds(ob, PCH)])
        gd.wait()
        pltpu.sync_copy(dbuf, drows_out.at[pl.ds(ob, PCH)])
        return carry

    lax.fori_loop(0, PNCH, body, 0)


_pairs_kernel = pl.kernel(
    _pairs_body,
    out_type=[
        jax.ShapeDtypeStruct((EL, D_H), jnp.float32),
        jax.ShapeDtypeStruct((EL, D_H), jnp.float32),
    ],
    mesh=_mesh,
    scratch_types=[
        pltpu.VMEM((PNCH, PCH), jnp.int32),
        pltpu.VMEM((PNCH, PCH), jnp.int32),
        pltpu.VMEM((PCH, D_H), jnp.float32),
        pltpu.VMEM((PCH, D_H), jnp.float32),
        pltpu.SemaphoreType.DMA,
        pltpu.SemaphoreType.DMA,
    ],
)


_BR = 1024


DEGS = 8


def _dinv_block(degp):
    deg = degp[0, :, 0:1] + degp[1, :, 0:1] + 1.0
    return lax.rsqrt(deg)


def _tc_hs1_body(x_ref, w_ref, degp_ref, out_ref):
    h = jnp.dot(x_ref[...], w_ref[...], preferred_element_type=jnp.float32)
    out_ref[...] = h * _dinv_block(degp_ref[...])


def _tc_mid_body(acc_ref, degp_ref, b1_ref, w2_ref, out_ref):
    dinv = _dinv_block(degp_ref[...])
    z1 = jnp.maximum((acc_ref[0] + acc_ref[1]) * dinv + b1_ref[...], 0.0)
    v = jnp.dot(z1, w2_ref[...], preferred_element_type=jnp.float32) * dinv
    out_ref[0] = v
    out_ref[1] = jnp.concatenate(
        [jnp.zeros((_BR, D_OUT), jnp.float32), v[:, :D_OUT]], axis=1)


def _tc_z2_body(acc_ref, degp_ref, hself_ref, b2_ref, out_ref):
    dinv = _dinv_block(degp_ref[...])
    v = (acc_ref[0] + acc_ref[1] + hself_ref[...]) * dinv + b2_ref[...]
    out_ref[...] = jnp.concatenate(
        [v, jnp.zeros((_BR, D_H - D_OUT), jnp.float32)], axis=1)


def _tc_dot_body(s_ref, d_ref, out_ref):
    out_ref[...] = jnp.sum(s_ref[...] * d_ref[...], axis=1, keepdims=True)


def _degp_spec():
    return pl.BlockSpec((NC, _BR, DEGS), lambda i: (0, i, 0))


def _row_spec(d):
    return pl.BlockSpec((_BR, d), lambda i: (i, 0))


def _acc_spec(d):
    return pl.BlockSpec((NC, _BR, d), lambda i: (0, i, 0))


def _full_spec(shape):
    return pl.BlockSpec(shape, lambda i: tuple(0 for _ in shape))


_tc_hs1 = pl.pallas_call(
    _tc_hs1_body,
    grid=(NP // _BR,),
    in_specs=[_row_spec(D_IN), _full_spec((D_IN, D_H)), _degp_spec()],
    out_specs=_row_spec(D_H),
    out_shape=jax.ShapeDtypeStruct((NP, D_H), jnp.float32),
)

_tc_mid = pl.pallas_call(
    _tc_mid_body,
    grid=(NP // _BR,),
    in_specs=[_acc_spec(D_H), _degp_spec(), _full_spec((1, D_H)),
              _full_spec((D_H, D_H))],
    out_specs=pl.BlockSpec((NC, _BR, D_H), lambda i: (0, i, 0)),
    out_shape=jax.ShapeDtypeStruct((NC, NP, D_H), jnp.float32),
)

_tc_z2 = pl.pallas_call(
    _tc_z2_body,
    grid=(NP // _BR,),
    in_specs=[_acc_spec(D_OUT), _degp_spec(), _row_spec(D_OUT),
              _full_spec((1, D_OUT))],
    out_specs=_row_spec(D_H),
    out_shape=jax.ShapeDtypeStruct((NP, D_H), jnp.float32),
)

_DBR = 2048

_tc_dot = pl.pallas_call(
    _tc_dot_body,
    grid=(EL // _DBR,),
    in_specs=[pl.BlockSpec((_DBR, D_H), lambda i: (i, 0)),
              pl.BlockSpec((_DBR, D_H), lambda i: (i, 0))],
    out_specs=pl.BlockSpec((_DBR, 1), lambda i: (i, 0)),
    out_shape=jax.ShapeDtypeStruct((EL, 1), jnp.float32),
)


@jax.jit
def kernel(x, edge_index, edge_label_index, W1, b1, W2, b2):
    pad_idx = N + jnp.arange(EP - E, dtype=jnp.int32) % (NP - N)
    epad = jnp.stack([pad_idx, pad_idx])
    ei_p = jnp.concatenate([edge_index, epad], axis=1)
    src_r = ei_p[0].reshape(NW, NCH, CH)
    dst_r = ei_p[1].reshape(NW, NCH, CH)
    els_r = edge_label_index[0].reshape(NW, PNCH, PCH)
    eld_r = edge_label_index[1].reshape(NW, PNCH, PCH)

    xp = jnp.concatenate([x, jnp.zeros((NP - N, D_IN), x.dtype)], axis=0)
    w2p = jnp.concatenate(
        [W2, jnp.zeros((D_H, D_H - D_OUT), W2.dtype)], axis=1)
    b2p = jnp.concatenate([b2, jnp.zeros((D_H - D_OUT,), b2.dtype)])

    zeros_h = jnp.zeros((NP, D_H), jnp.float32)
    ones_ch = jnp.ones((CH, D_H), jnp.float32)

    degp = _deg_kernel(dst_r, zeros_h, ones_ch)
    degs = degp[:, :, :DEGS]

    hs1 = _tc_hs1(xp, W1, degs)
    acc1 = _agg_kernel(hs1, src_r, dst_r, zeros_h)
    hs2d = _tc_mid(acc1, degs, b1.reshape(1, D_H), w2p)
    g2 = (ei_p[0] + NP * (ei_p[1] & 1)).reshape(NW, NCH, CH)
    d2 = (ei_p[1] >> 1).reshape(NW, NCH, CH)
    zeros_half = jnp.zeros((NH, D_H), jnp.float32)
    acc2p = _agg2_kernel(hs2d.reshape(2 * NP, D_H), g2, d2, zeros_half)
    acc2 = acc2p.reshape(NC, NP, D_OUT)
    hs2self = hs2d[0, :, :D_OUT]
    z2 = _tc_z2(acc2, degs, hs2self, b2p[:D_OUT].reshape(1, D_OUT))

    srows, drows = _pairs_kernel(z2, els_r, eld_r)
    return _tc_dot(srows, drows).reshape(-1)


# speedup vs baseline: 2.1136x; 1.0019x over previous
import jax
import jax.numpy as jnp
from jax import lax
from jax.experimental import pallas as pl
from jax.experimental.pallas import tpu as pltpu
from jax.experimental.pallas import tpu_sc as plsc

N = 10000
NP = 10240
E = 320000
EL = 16384
D_IN = 128
D_H = 128
D_OUT = 64

NC = 2
NS = 16
NW = NC * NS
RP = NP // NS

EP = 327680
EPT = EP // NW
CH = 128
NCH = EPT // CH

NH = NP // 2

PPT = EL // NW
PCH = 128
PNCH = PPT // PCH

_mesh = plsc.VectorSubcoreMesh(
    core_axis_name="c", subcore_axis_name="s", num_cores=NC, num_subcores=NS
)


def _wid():
    return lax.axis_index("s") * NC + lax.axis_index("c")


def _rbase():
    return pl.multiple_of(lax.axis_index("s") * RP, 8)


def _deg_body(dst_hbm, zeros_hbm, ones_hbm, deg_out, idx_v, ones_v, deg_sp, sem):
    c = lax.axis_index("c")
    rb = _rbase()
    w = _wid()
    pltpu.sync_copy(zeros_hbm.at[pl.ds(rb, RP)], deg_sp.at[pl.ds(rb, RP)])
    pltpu.sync_copy(ones_hbm, ones_v)
    pltpu.sync_copy(dst_hbm.at[w], idx_v)
    plsc.subcore_barrier()

    def body(j, carry):
        pltpu.sync_copy(ones_v, deg_sp.at[idx_v.at[j]], add=True)
        return carry

    lax.fori_loop(0, NCH, body, 0)
    plsc.subcore_barrier()
    pltpu.sync_copy(deg_sp.at[pl.ds(rb, RP)], deg_out.at[c, pl.ds(rb, RP)])


_deg_kernel = pl.kernel(
    _deg_body,
    out_type=jax.ShapeDtypeStruct((NC, NP, D_H), jnp.float32),
    mesh=_mesh,
    scratch_types=[
        pltpu.VMEM((NCH, CH), jnp.int32),
        pltpu.VMEM((CH, D_H), jnp.float32),
        pltpu.VMEM_SHARED((NP, D_H), jnp.float32),
        pltpu.SemaphoreType.DMA,
    ],
)


def _agg_body(hs_hbm, src_hbm, dst_hbm, zeros_hbm, acc_out,
              sidx_v, didx_v, rows0, acc_sp, gsem0):
    c = lax.axis_index("c")
    rb = _rbase()
    w = _wid()

    @pl.when(c == 0)
    def _():
        pltpu.sync_copy(hs_hbm.at[pl.ds(rb, RP)], acc_sp.at[pl.ds(rb, RP)])

    @pl.when(c != 0)
    def _():
        pltpu.sync_copy(zeros_hbm.at[pl.ds(rb, RP)], acc_sp.at[pl.ds(rb, RP)])

    pltpu.sync_copy(src_hbm.at[w], sidx_v)
    pltpu.sync_copy(dst_hbm.at[w], didx_v)
    plsc.subcore_barrier()

    def body(j, carry):
        pltpu.async_copy(hs_hbm.at[sidx_v.at[j]], rows0, gsem0).wait()
        pltpu.sync_copy(rows0, acc_sp.at[didx_v.at[j]], add=True)
        return carry

    lax.fori_loop(0, NCH, body, 0)
    plsc.subcore_barrier()
    pltpu.sync_copy(acc_sp.at[pl.ds(rb, RP)], acc_out.at[c, pl.ds(rb, RP)])


_agg_kernel = pl.kernel(
    _agg_body,
    out_type=jax.ShapeDtypeStruct((NC, NP, D_H), jnp.float32),
    mesh=_mesh,
    scratch_types=[
        pltpu.VMEM((NCH, CH), jnp.int32),
        pltpu.VMEM((NCH, CH), jnp.int32),
        pltpu.VMEM((CH, D_H), jnp.float32),
        pltpu.VMEM_SHARED((NP, D_H), jnp.float32),
        pltpu.SemaphoreType.DMA,
    ],
)


def _agg2_body(hs_hbm, src_hbm, dst_hbm, zeros_hbm, acc_out,
               sidx_v, didx_v, rows0, rows1, rows2, rows3, acc_sp,
               gsem0, gsem1, gsem2, gsem3):
    c = lax.axis_index("c")
    s = lax.axis_index("s")
    rb = pl.multiple_of(s * (NH // NS), 8)
    w = _wid()
    hp = NH // NS

    pltpu.sync_copy(zeros_hbm.at[pl.ds(rb, hp)], acc_sp.at[pl.ds(rb, hp)])
    pltpu.sync_copy(src_hbm.at[w], sidx_v)
    pltpu.sync_copy(dst_hbm.at[w], didx_v)
    plsc.subcore_barrier()

    def body(jj, carry):
        t = jj * 4
        g0 = pltpu.async_copy(hs_hbm.at[sidx_v.at[t]], rows0, gsem0)
        g1 = pltpu.async_copy(hs_hbm.at[sidx_v.at[t + 1]], rows1, gsem1)
        g2 = pltpu.async_copy(hs_hbm.at[sidx_v.at[t + 2]], rows2, gsem2)
        g3 = pltpu.async_copy(hs_hbm.at[sidx_v.at[t + 3]], rows3, gsem3)
        g0.wait()
        pltpu.sync_copy(rows0, acc_sp.at[didx_v.at[t]], add=True)
        g1.wait()
        pltpu.sync_copy(rows1, acc_sp.at[didx_v.at[t + 1]], add=True)
        g2.wait()
        pltpu.sync_copy(rows2, acc_sp.at[didx_v.at[t + 2]], add=True)
        g3.wait()
        pltpu.sync_copy(rows3, acc_sp.at[didx_v.at[t + 3]], add=True)
        return carry

    lax.fori_loop(0, NCH // 4, body, 0)
    plsc.subcore_barrier()
    pltpu.sync_copy(acc_sp.at[pl.ds(rb, hp)], acc_out.at[c, pl.ds(rb, hp)])


_agg2_kernel = pl.kernel(
    _agg2_body,
    out_type=jax.ShapeDtypeStruct((NC, NH, D_H), jnp.float32),
    mesh=_mesh,
    scratch_types=[
        pltpu.VMEM((NCH, CH), jnp.int32),
        pltpu.VMEM((NCH, CH), jnp.int32),
        pltpu.VMEM((CH, D_H), jnp.float32),
        pltpu.VMEM((CH, D_H), jnp.float32),
        pltpu.VMEM((CH, D_H), jnp.float32),
        pltpu.VMEM((CH, D_H), jnp.float32),
        pltpu.VMEM_SHARED((NH, D_H), jnp.float32),
        pltpu.SemaphoreType.DMA,
        pltpu.SemaphoreType.DMA,
        pltpu.SemaphoreType.DMA,
        pltpu.SemaphoreType.DMA,
    ],
)


def _pairs_body(z_hbm, sidx_hbm, didx_hbm, srows_out, drows_out,
                sidx_v, didx_v, sbuf, dbuf, sem0, sem1):
    w = _wid()
    pltpu.sync_copy(sidx_hbm.at[w], sidx_v)
    pltpu.sync_copy(didx_hbm.at[w], didx_v)

    def body(j, carry):
        ob = pl.multiple_of(w * PPT + j * PCH, 8)
        gs = pltpu.async_copy(z_hbm.at[sidx_v.at[j]], sbuf, sem0)
        gd = pltpu.async_copy(z_hbm.at[didx_v.at[j]], dbuf, sem1)
        gs.wait()
        ws = pltpu.async_copy(sbuf, srows_out.at[pl.ds(ob, PCH)], sem0)
        gd.wait()
        wd = pltpu.async_copy(dbuf, drows_out.at[pl.ds(ob, PCH)], sem1)
        ws.wait()
        wd.wait()
        return carry

    lax.fori_loop(0, PNCH, body, 0)


_pairs_kernel = pl.kernel(
    _pairs_body,
    out_type=[
        jax.ShapeDtypeStruct((EL, D_H), jnp.float32),
        jax.ShapeDtypeStruct((EL, D_H), jnp.float32),
    ],
    mesh=_mesh,
    scratch_types=[
        pltpu.VMEM((PNCH, PCH), jnp.int32),
        pltpu.VMEM((PNCH, PCH), jnp.int32),
        pltpu.VMEM((PCH, D_H), jnp.float32),
        pltpu.VMEM((PCH, D_H), jnp.float32),
        pltpu.SemaphoreType.DMA,
        pltpu.SemaphoreType.DMA,
    ],
)


_BR = 1024


DEGS = 8


def _dinv_block(degp):
    deg = degp[0, :, 0:1] + degp[1, :, 0:1] + 1.0
    return lax.rsqrt(deg)


def _tc_hs1_body(x_ref, w_ref, degp_ref, out_ref):
    h = jnp.dot(x_ref[...], w_ref[...], preferred_element_type=jnp.float32)
    out_ref[...] = h * _dinv_block(degp_ref[...])


def _tc_mid_body(acc_ref, degp_ref, b1_ref, w2_ref, out_ref):
    dinv = _dinv_block(degp_ref[...])
    z1 = jnp.maximum((acc_ref[0] + acc_ref[1]) * dinv + b1_ref[...], 0.0)
    v = jnp.dot(z1, w2_ref[...], preferred_element_type=jnp.float32) * dinv
    out_ref[0] = v
    out_ref[1] = jnp.concatenate(
        [jnp.zeros((_BR, D_OUT), jnp.float32), v[:, :D_OUT]], axis=1)


def _tc_z2_body(acc_ref, degp_ref, hself_ref, b2_ref, out_ref):
    dinv = _dinv_block(degp_ref[...])
    v = (acc_ref[0] + acc_ref[1] + hself_ref[...]) * dinv + b2_ref[...]
    out_ref[...] = jnp.concatenate(
        [v, jnp.zeros((_BR, D_H - D_OUT), jnp.float32)], axis=1)


def _tc_dot_body(s_ref, d_ref, out_ref):
    out_ref[...] = jnp.sum(s_ref[...] * d_ref[...], axis=1, keepdims=True)


def _degp_spec():
    return pl.BlockSpec((NC, _BR, DEGS), lambda i: (0, i, 0))


def _row_spec(d):
    return pl.BlockSpec((_BR, d), lambda i: (i, 0))


def _acc_spec(d):
    return pl.BlockSpec((NC, _BR, d), lambda i: (0, i, 0))


def _full_spec(shape):
    return pl.BlockSpec(shape, lambda i: tuple(0 for _ in shape))


_tc_hs1 = pl.pallas_call(
    _tc_hs1_body,
    grid=(NP // _BR,),
    in_specs=[_row_spec(D_IN), _full_spec((D_IN, D_H)), _degp_spec()],
    out_specs=_row_spec(D_H),
    out_shape=jax.ShapeDtypeStruct((NP, D_H), jnp.float32),
)

_tc_mid = pl.pallas_call(
    _tc_mid_body,
    grid=(NP // _BR,),
    in_specs=[_acc_spec(D_H), _degp_spec(), _full_spec((1, D_H)),
              _full_spec((D_H, D_H))],
    out_specs=pl.BlockSpec((NC, _BR, D_H), lambda i: (0, i, 0)),
    out_shape=jax.ShapeDtypeStruct((NC, NP, D_H), jnp.float32),
)

_tc_z2 = pl.pallas_call(
    _tc_z2_body,
    grid=(NP // _BR,),
    in_specs=[_acc_spec(D_OUT), _degp_spec(), _row_spec(D_OUT),
              _full_spec((1, D_OUT))],
    out_specs=_row_spec(D_H),
    out_shape=jax.ShapeDtypeStruct((NP, D_H), jnp.float32),
)

_DBR = 2048

_tc_dot = pl.pallas_call(
    _tc_dot_body,
    grid=(EL // _DBR,),
    in_specs=[pl.BlockSpec((_DBR, D_H), lambda i: (i, 0)),
              pl.BlockSpec((_DBR, D_H), lambda i: (i, 0))],
    out_specs=pl.BlockSpec((_DBR, 1), lambda i: (i, 0)),
    out_shape=jax.ShapeDtypeStruct((EL, 1), jnp.float32),
)


@jax.jit
def kernel(x, edge_index, edge_label_index, W1, b1, W2, b2):
    pad_idx = N + jnp.arange(EP - E, dtype=jnp.int32) % (NP - N)
    epad = jnp.stack([pad_idx, pad_idx])
    ei_p = jnp.concatenate([edge_index, epad], axis=1)
    src_r = ei_p[0].reshape(NW, NCH, CH)
    dst_r = ei_p[1].reshape(NW, NCH, CH)
    els_r = edge_label_index[0].reshape(NW, PNCH, PCH)
    eld_r = edge_label_index[1].reshape(NW, PNCH, PCH)

    xp = jnp.concatenate([x, jnp.zeros((NP - N, D_IN), x.dtype)], axis=0)
    w2p = jnp.concatenate(
        [W2, jnp.zeros((D_H, D_H - D_OUT), W2.dtype)], axis=1)
    b2p = jnp.concatenate([b2, jnp.zeros((D_H - D_OUT,), b2.dtype)])

    zeros_h = jnp.zeros((NP, D_H), jnp.float32)
    ones_ch = jnp.ones((CH, D_H), jnp.float32)

    degp = _deg_kernel(dst_r, zeros_h, ones_ch)
    degs = degp[:, :, :DEGS]

    hs1 = _tc_hs1(xp, W1, degs)
    acc1 = _agg_kernel(hs1, src_r, dst_r, zeros_h)
    hs2d = _tc_mid(acc1, degs, b1.reshape(1, D_H), w2p)
    g2 = (ei_p[0] + NP * (ei_p[1] & 1)).reshape(NW, NCH, CH)
    d2 = (ei_p[1] >> 1).reshape(NW, NCH, CH)
    zeros_half = jnp.zeros((NH, D_H), jnp.float32)
    acc2p = _agg2_kernel(hs2d.reshape(2 * NP, D_H), g2, d2, zeros_half)
    acc2 = acc2p.reshape(NC, NP, D_OUT)
    hs2self = hs2d[0, :, :D_OUT]
    z2 = _tc_z2(acc2, degs, hs2self, b2p[:D_OUT].reshape(1, D_OUT))

    srows, drows = _pairs_kernel(z2, els_r, eld_r)
    return _tc_dot(srows, drows).reshape(-1)
